# baseline (speedup 1.0000x reference)
import functools

import jax
import jax.numpy as jnp
from jax import lax
from jax.experimental import pallas as pl
from jax.experimental.pallas import tpu as pltpu
from jax.experimental.pallas import tpu_sc as plsc

N = 10000
E = 320000
G = 64
AIF = 92
BINS = 32
FC = 128

NC = 2
NS = 16
NW = NC * NS
EPW = E // NW
CH = 40
NCH = EPW // CH
EH = E // 2
EPWH = EH // NW
CHG = 40
NCHG = EPWH // CHG
ROWS_PER_TILE = 624
ROWS_TAIL = N - ROWS_PER_TILE * NS

EB = 8000
NEB = E // EB
NEBH = EH // EB


@functools.lru_cache(maxsize=None)
def _sc_mesh():
    return plsc.VectorSubcoreMesh(core_axis_name="c", subcore_axis_name="s",
                                  num_cores=NC, num_subcores=NS)


K = 5
NG = NCHG // K
KS = 5
NGS = NCH // KS


@functools.lru_cache(maxsize=None)
def _build_gather_sc():
    half_out = jax.ShapeDtypeStruct((EH, FC), jnp.float32)

    @functools.partial(
        pl.kernel,
        out_type=[half_out, half_out, half_out, half_out],
        mesh=_sc_mesh(),
        scratch_types=[
            [pltpu.VMEM((NCHG, CHG), jnp.int32)] * 4,
            [pltpu.VMEM((CHG, FC), jnp.float32)] * K,
            [pltpu.VMEM((CHG, FC), jnp.float32)] * K,
            [pltpu.SemaphoreType.DMA] * K,
            [pltpu.SemaphoreType.DMA] * K,
            pltpu.SemaphoreType.DMA,
        ],
    )
    def gather_sc(nf_hbm, dst_a, src_a, dst_b, src_b,
                  xi_a, xj_a, xi_b, xj_b,
                  idxs, bufs_i, bufs_j, sems_g, sems_w, sem0):
        wid = lax.axis_index("s") * NC + lax.axis_index("c")
        base = wid * EPWH
        stage = []
        for src_ref, idx_ref in zip((dst_a, src_a, dst_b, src_b), idxs):
            stage.append(pltpu.async_copy(src_ref.at[wid], idx_ref, sem0))
        for cp in stage:
            cp.wait()

        def run_half(idx_d, idx_s, xi_hbm, xj_hbm):
            def body(g, carry):
                t0 = g * K
                fired = []
                for b in range(K):
                    t = t0 + b
                    off = base + t * CHG

                    @pl.when(g > 0)
                    def _(b=b, off=off):
                        pltpu.make_async_copy(
                            bufs_i[b], xi_hbm.at[pl.ds(off, CHG)],
                            sems_w[b]).wait()
                        pltpu.make_async_copy(
                            bufs_j[b], xj_hbm.at[pl.ds(off, CHG)],
                            sems_w[b]).wait()

                    ci = pltpu.async_copy(nf_hbm.at[idx_d.at[t]], bufs_i[b],
                                          sems_g[b])
                    cj = pltpu.async_copy(nf_hbm.at[idx_s.at[t]], bufs_j[b],
                                          sems_g[b])
                    fired.append((ci, cj))
                for b in range(K):
                    t = t0 + b
                    off = base + t * CHG
                    ci, cj = fired[b]
                    ci.wait()
                    cj.wait()
                    pltpu.async_copy(bufs_i[b], xi_hbm.at[pl.ds(off, CHG)],
                                     sems_w[b])
                    pltpu.async_copy(bufs_j[b], xj_hbm.at[pl.ds(off, CHG)],
                                     sems_w[b])
                return carry

            lax.fori_loop(0, NG, body, 0)
            for b in range(K):
                off = base + ((NG - 1) * K + b) * CHG
                pltpu.make_async_copy(
                    bufs_i[b], xi_hbm.at[pl.ds(off, CHG)], sems_w[b]).wait()
                pltpu.make_async_copy(
                    bufs_j[b], xj_hbm.at[pl.ds(off, CHG)], sems_w[b]).wait()

        run_half(idxs[0], idxs[1], xi_a, xj_a)
        run_half(idxs[2], idxs[3], xi_b, xj_b)

    return gather_sc


@functools.lru_cache(maxsize=None)
def _build_gather_half_sc():
    half_out = jax.ShapeDtypeStruct((EH, FC), jnp.float32)

    @functools.partial(
        pl.kernel,
        out_type=[half_out, half_out],
        mesh=_sc_mesh(),
        scratch_types=[
            [pltpu.VMEM((NCHG, CHG), jnp.int32)] * 2,
            [pltpu.VMEM((CHG, FC), jnp.float32)] * K,
            [pltpu.VMEM((CHG, FC), jnp.float32)] * K,
            [pltpu.SemaphoreType.DMA] * K,
            [pltpu.SemaphoreType.DMA] * K,
            pltpu.SemaphoreType.DMA,
        ],
    )
    def gather_half(nf_hbm, dst3, src3, xi_hbm, xj_hbm,
                    idxs, bufs_i, bufs_j, sems_g, sems_w, sem0):
        wid = lax.axis_index("s") * NC + lax.axis_index("c")
        base = wid * EPWH
        cp = pltpu.async_copy(dst3.at[wid], idxs[0], sem0)
        cp2 = pltpu.async_copy(src3.at[wid], idxs[1], sem0)
        cp.wait()
        cp2.wait()

        def body(g, carry):
            t0 = g * K
            fired = []
            for b in range(K):
                t = t0 + b
                off = base + t * CHG

                @pl.when(g > 0)
                def _(b=b, off=off):
                    pltpu.make_async_copy(
                        bufs_i[b], xi_hbm.at[pl.ds(off, CHG)],
                        sems_w[b]).wait()
                    pltpu.make_async_copy(
                        bufs_j[b], xj_hbm.at[pl.ds(off, CHG)],
                        sems_w[b]).wait()

                ci = pltpu.async_copy(nf_hbm.at[idxs[0].at[t]], bufs_i[b],
                                      sems_g[b])
                cj = pltpu.async_copy(nf_hbm.at[idxs[1].at[t]], bufs_j[b],
                                      sems_g[b])
                fired.append((ci, cj))
            for b in range(K):
                t = t0 + b
                off = base + t * CHG
                ci, cj = fired[b]
                ci.wait()
                cj.wait()
                pltpu.async_copy(bufs_i[b], xi_hbm.at[pl.ds(off, CHG)],
                                 sems_w[b])
                pltpu.async_copy(bufs_j[b], xj_hbm.at[pl.ds(off, CHG)],
                                 sems_w[b])
            return carry

        lax.fori_loop(0, NG, body, 0)
        for b in range(K):
            off = base + ((NG - 1) * K + b) * CHG
            pltpu.make_async_copy(
                bufs_i[b], xi_hbm.at[pl.ds(off, CHG)], sems_w[b]).wait()
            pltpu.make_async_copy(
                bufs_j[b], xj_hbm.at[pl.ds(off, CHG)], sems_w[b]).wait()

    return gather_half


def _gather_sc(nf, dst_a, src_a, dst_b, src_b):
    k = _build_gather_half_sc()
    xia, xja = k(nf, dst_a, src_a)
    xib, xjb = k(nf, dst_b, src_b)
    return xia, xja, xib, xjb


NGSH = EH // NW // (KS * CH)


@functools.lru_cache(maxsize=None)
def _build_scatter_sc(half):
    init_shape = (jax.ShapeDtypeStruct((N, FC), jnp.float32) if half == 0
                  else jax.ShapeDtypeStruct((NC, N, FC), jnp.float32))

    @functools.partial(
        pl.kernel,
        out_type=jax.ShapeDtypeStruct((NC, N, FC), jnp.float32),
        mesh=_sc_mesh(),
        scratch_types=[
            [pltpu.VMEM((KS, CH), jnp.int32)] * 2,
            [pltpu.VMEM((CH, FC), jnp.float32)] * KS,
            [pltpu.SemaphoreType.DMA] * KS,
            [pltpu.SemaphoreType.DMA] * KS,
            [pltpu.SemaphoreType.DMA] * 2,
            pltpu.VMEM_SHARED((N, FC), jnp.float32),
        ],
    )
    def scatter_sc(m_hbm, dst4_hbm, init_hbm, out_hbm, idx_bufs, bufs,
                   sems_l, sems_a, sems_i, acc_sh):
        cid = lax.axis_index("c")
        sid = lax.axis_index("s")
        wid = sid * NC + cid
        base = half * EH + wid * (EH // NW)

        row0 = sid * ROWS_PER_TILE
        src0 = (init_hbm.at[pl.ds(row0, ROWS_PER_TILE)] if half == 0
                else init_hbm.at[cid, pl.ds(row0, ROWS_PER_TILE)])
        pltpu.sync_copy(src0, acc_sh.at[pl.ds(row0, ROWS_PER_TILE)])

        @pl.when(sid == 0)
        def _():
            tail0 = ROWS_PER_TILE * NS
            srct = (init_hbm.at[pl.ds(tail0, ROWS_TAIL)] if half == 0
                    else init_hbm.at[cid, pl.ds(tail0, ROWS_TAIL)])
            pltpu.sync_copy(srct, acc_sh.at[pl.ds(tail0, ROWS_TAIL)])

        pltpu.async_copy(dst4_hbm.at[wid, 0], idx_bufs[0], sems_i[0]).wait()
        plsc.subcore_barrier()

        def group(g, sel, nsel):
            for b in range(KS):
                @pl.when(g > 0)
                def _(b=b):
                    pltpu.make_async_copy(
                        bufs[b], acc_sh.at[idx_bufs[sel].at[b]],
                        sems_a[b]).wait()

            @pl.when(g + 1 < NGSH)
            def _():
                pltpu.async_copy(dst4_hbm.at[wid, g + 1], idx_bufs[nsel],
                                 sems_i[nsel])

            t0 = g * KS
            fired = []
            for b in range(KS):
                off = base + (t0 + b) * CH
                fired.append(pltpu.async_copy(
                    m_hbm.at[pl.ds(off, CH)], bufs[b], sems_l[b]))
            for b in range(KS):
                fired[b].wait()
                pltpu.async_copy(bufs[b], acc_sh.at[idx_bufs[sel].at[b]],
                                 sems_a[b], add=True)

            @pl.when(g + 1 < NGSH)
            def _():
                pltpu.make_async_copy(dst4_hbm.at[wid, g + 1],
                                      idx_bufs[nsel], sems_i[nsel]).wait()

        def body(gg, carry):
            group(2 * gg, 0, 1)
            group(2 * gg + 1, 1, 0)
            return carry

        lax.fori_loop(0, NGSH // 2, body, 0)
        group(NGSH - 1, (NGSH - 1) % 2, NGSH % 2)
        last_sel = (NGSH - 1) % 2
        for b in range(KS):
            pltpu.make_async_copy(
                bufs[b], acc_sh.at[idx_bufs[last_sel].at[b]],
                sems_a[b]).wait()

        plsc.subcore_barrier()
        pltpu.sync_copy(acc_sh.at[pl.ds(row0, ROWS_PER_TILE)],
                        out_hbm.at[cid, pl.ds(row0, ROWS_PER_TILE)])

        @pl.when(sid == 0)
        def _():
            tail0 = ROWS_PER_TILE * NS
            pltpu.sync_copy(acc_sh.at[pl.ds(tail0, ROWS_TAIL)],
                            out_hbm.at[cid, pl.ds(tail0, ROWS_TAIL)])

    return scatter_sc


def _scatter_sc(m, dst4a, dst4b, zeros):
    p = _build_scatter_sc(0)(m, dst4a, zeros)
    return _build_scatter_sc(1)(m, dst4b, p)


def _dot(a, b):
    return jnp.dot(a, b, preferred_element_type=jnp.float32)


def _prep_body(x_ref, w_ref, b_ref, nf_ref):
    nf_ref[...] = _dot(x_ref[...], w_ref[...]) + b_ref[...]


def _prep_tc(x, w_atom, b_atom):
    return pl.pallas_call(
        _prep_body,
        out_shape=jax.ShapeDtypeStruct((N, FC), jnp.float32),
    )(x, w_atom, b_atom)


def _edge_h(ea, xi, xj, wei, bei, weu, beu, wemb, bemb):
    centers = (lax.broadcasted_iota(jnp.int32, (1, BINS), 1)
               .astype(jnp.float32) * (8.0 / (BINS - 1)))
    gamma = (BINS - 1) / 8.0
    r = jnp.exp(-gamma * (ea[:, None] - centers) ** 2)
    a1, a2, a3 = wei[0:FC], wei[FC:2 * FC], wei[2 * FC:3 * FC]
    c1, c2, c3 = weu[0:FC], weu[FC:2 * FC], weu[2 * FC:3 * FC]
    weg = _dot(wemb, a3)
    weu_g = _dot(wemb, c3)
    cg = bei + _dot(bemb, a3)
    cu = beu + _dot(bemb, c3)
    hg = _dot(xi, a1) + _dot(xj, a2) + _dot(r, weg) + cg
    hu = _dot(xi, c1) + _dot(xj, c2) + _dot(r, weu_g) + cu
    return hg, hu


def _stats_body(ea_ref, xi_ref, xj_ref, wei_ref, bei_ref, weu_ref, beu_ref,
                wemb_ref, bemb_ref, stats_ref):
    hg, hu = _edge_h(ea_ref[0, 0, :], xi_ref[...], xj_ref[...],
                     wei_ref[...], bei_ref[...], weu_ref[...], beu_ref[...],
                     wemb_ref[...], bemb_ref[...])
    blk = jnp.stack([
        jnp.sum(hg, axis=0), jnp.sum(hg * hg, axis=0),
        jnp.sum(hu, axis=0), jnp.sum(hu * hu, axis=0),
    ], axis=0)

    @pl.when(pl.program_id(0) == 0)
    def _():
        stats_ref[...] = jnp.zeros_like(stats_ref)

    stats_ref[...] += blk


def _edge_stats_tc(ea3, xi, xj, wei, bei, weu, beu, wemb, bemb):
    wspec = pl.BlockSpec
    return pl.pallas_call(
        _stats_body,
        grid=(NEBH,),
        in_specs=[
            wspec((1, 1, EB), lambda i: (i, 0, 0)),
            wspec((EB, FC), lambda i: (i, 0)),
            wspec((EB, FC), lambda i: (i, 0)),
            wspec((3 * FC, FC), lambda i: (0, 0)),
            wspec((1, FC), lambda i: (0, 0)),
            wspec((3 * FC, FC), lambda i: (0, 0)),
            wspec((1, FC), lambda i: (0, 0)),
            wspec((BINS, FC), lambda i: (0, 0)),
            wspec((1, FC), lambda i: (0, 0)),
        ],
        out_specs=wspec((4, FC), lambda i: (0, 0)),
        out_shape=jax.ShapeDtypeStruct((4, FC), jnp.float32),
    )(ea3, xi, xj, wei, bei, weu, beu, wemb, bemb)


def _sigmoid(v):
    return 1.0 / (1.0 + jnp.exp(-v))


def _softplus(v):
    return jnp.maximum(v, 0.0) + jnp.log1p(jnp.exp(-jnp.abs(v)))


def _m_body(m_in_ref, ea_ref, xi_ref, xj_ref, sa_ref, sb_ref, wei_ref,
            bei_ref, weu_ref, beu_ref, wemb_ref, bemb_ref, gg_ref, gb_ref,
            ug_ref, ub_ref, m_ref):
    del m_in_ref
    hg, hu = _edge_h(ea_ref[0, 0, :], xi_ref[...], xj_ref[...],
                     wei_ref[...], bei_ref[...], weu_ref[...], beu_ref[...],
                     wemb_ref[...], bemb_ref[...])
    s = sa_ref[...] + sb_ref[...]
    inv_e = 1.0 / E
    mu_g = s[0:1, :] * inv_e
    var_g = s[1:2, :] * inv_e - mu_g * mu_g
    mu_u = s[2:3, :] * inv_e
    var_u = s[3:4, :] * inv_e - mu_u * mu_u
    sc_g = gg_ref[...] * lax.rsqrt(var_g + 1e-5)
    sc_u = ug_ref[...] * lax.rsqrt(var_u + 1e-5)
    gate = _sigmoid((hg - mu_g) * sc_g + gb_ref[...])
    upd = _softplus((hu - mu_u) * sc_u + ub_ref[...])
    m_ref[...] = gate * upd


def _edge_m_tc(half, m_in, ea3, xi, xj, sa, sb, wei, bei, weu, beu,
               wemb, bemb, gg, gb, ug, ub):
    wspec = pl.BlockSpec
    base = half * NEBH
    if m_in is None:
        body = functools.partial(_m_body, None)
        first_specs, first_args, aliases = [], (), {}
    else:
        body = _m_body
        first_specs = [wspec(memory_space=pl.ANY)]
        first_args = (m_in,)
        aliases = {0: 0}
    return pl.pallas_call(
        body,
        grid=(NEBH,),
        in_specs=first_specs + [
            wspec((1, 1, EB), lambda i: (i, 0, 0)),
            wspec((EB, FC), lambda i: (i, 0)),
            wspec((EB, FC), lambda i: (i, 0)),
            wspec((4, FC), lambda i: (0, 0)),
            wspec((4, FC), lambda i: (0, 0)),
            wspec((3 * FC, FC), lambda i: (0, 0)),
            wspec((1, FC), lambda i: (0, 0)),
            wspec((3 * FC, FC), lambda i: (0, 0)),
            wspec((1, FC), lambda i: (0, 0)),
            wspec((BINS, FC), lambda i: (0, 0)),
            wspec((1, FC), lambda i: (0, 0)),
            wspec((1, FC), lambda i: (0, 0)),
            wspec((1, FC), lambda i: (0, 0)),
            wspec((1, FC), lambda i: (0, 0)),
            wspec((1, FC), lambda i: (0, 0)),
        ],
        out_specs=wspec((EB, FC), lambda i: (i + base, 0)),
        out_shape=jax.ShapeDtypeStruct((E, FC), jnp.float32),
        input_output_aliases=aliases,
    )(*first_args, ea3, xi, xj, sa, sb, wei, bei, weu, beu, wemb, bemb,
      gg, gb, ug, ub)


def _final_body(nf_ref, aggp_ref, batch_ref, wat_ref, bat_ref, bng_ref,
                bnb_ref, wfc_ref, bfc_ref, wout_ref, bout_ref, out_ref):
    agg = aggp_ref[0] + aggp_ref[1]
    mu = jnp.mean(agg, axis=0, keepdims=True)
    var = jnp.mean(agg * agg, axis=0, keepdims=True) - mu * mu
    bn = (agg - mu) * lax.rsqrt(var + 1e-5) * bng_ref[...] + bnb_ref[...]
    nf2 = _dot(nf_ref[...] + bn, wat_ref[...]) + bat_ref[...]

    gids = lax.broadcasted_iota(jnp.int32, (1, G), 1)
    onehot = (batch_ref[...] == gids).astype(jnp.float32)
    counts = jnp.sum(onehot, axis=0)
    pooled_sum = lax.dot_general(onehot, nf2, (((0,), (0,)), ((), ())),
                                 preferred_element_type=jnp.float32)
    pooled = pooled_sum / jnp.maximum(counts, 1.0)[:, None]
    feats = _dot(pooled, wfc_ref[...]) + bfc_ref[...]
    feats = feats * _sigmoid(feats)
    out_ref[...] = _dot(feats, wout_ref[...]) + bout_ref[...]


def _final_tc(nf, aggp, batch2, wat, bat, bng, bnb, wfc, bfc, wout, bout):
    return pl.pallas_call(
        _final_body,
        out_shape=jax.ShapeDtypeStruct((G, 1), jnp.float32),
    )(nf, aggp, batch2, wat, bat, bng, bnb, wfc, bfc, wout, bout)


def kernel(x, edge_attr, edge_index, batch,
           W_atom_emb, b_atom_emb, W_edge_emb, b_edge_emb,
           W_atoms, b_atoms, bn_g, bn_b,
           W_ei, b_ei, bn_ei_g, bn_ei_b,
           W_eu, b_eu, bn_eu_g, bn_eu_b,
           W_fc, b_fc, W_out, b_out):
    src = edge_index[0]
    dst = edge_index[1]
    dst4a = lax.slice_in_dim(dst, 0, EH).reshape(NW, NGSH, KS, CH)
    dst4b = lax.slice_in_dim(dst, EH, E).reshape(NW, NGSH, KS, CH)
    halves = []
    for h in range(2):
        d3 = lax.slice_in_dim(dst, h * EH, (h + 1) * EH).reshape(
            NW, NCHG, CHG)
        s3 = lax.slice_in_dim(src, h * EH, (h + 1) * EH).reshape(
            NW, NCHG, CHG)
        ea3 = lax.slice_in_dim(edge_attr, h * EH, (h + 1) * EH).reshape(
            NEBH, 1, EB)
        halves.append((d3, s3, ea3))
    r1 = lambda v: v.reshape(1, FC)

    nf = _prep_tc(x, W_atom_emb, b_atom_emb.reshape(1, FC))
    xia, xja, xib, xjb = _gather_sc(nf, halves[0][0], halves[0][1],
                                    halves[1][0], halves[1][1])
    gathered = [(xia, xja), (xib, xjb)]

    wargs = (W_ei, r1(b_ei), W_eu, r1(b_eu), W_edge_emb, r1(b_edge_emb))
    stats = [_edge_stats_tc(halves[h][2], *gathered[h], *wargs)
             for h in range(2)]

    bnargs = (r1(bn_ei_g), r1(bn_ei_b), r1(bn_eu_g), r1(bn_eu_b))
    m = None
    for h in range(2):
        m = _edge_m_tc(h, m, halves[h][2], *gathered[h], stats[0], stats[1],
                       *wargs, *bnargs)

    zeros = jnp.zeros((N, FC), jnp.float32)
    aggp = _scatter_sc(m, dst4a, dst4b, zeros)

    out = _final_tc(nf, aggp, batch.reshape(N, 1), W_atoms,
                    r1(b_atoms), r1(bn_g), r1(bn_b), W_fc, r1(b_fc),
                    W_out, b_out.reshape(1, 1))
    return jnp.squeeze(out)


# speedup vs baseline: 1.0202x; 1.0202x over previous
import functools

import jax
import jax.numpy as jnp
from jax import lax
from jax.experimental import pallas as pl
from jax.experimental.pallas import tpu as pltpu
from jax.experimental.pallas import tpu_sc as plsc

N = 10000
E = 320000
G = 64
AIF = 92
BINS = 32
FC = 128

NC = 2
NS = 16
NW = NC * NS
EPW = E // NW
CH = 40
NCH = EPW // CH
EH = E // 2
EPWH = EH // NW
CHG = 40
NCHG = EPWH // CHG
ROWS_PER_TILE = 624
ROWS_TAIL = N - ROWS_PER_TILE * NS

EB = 8000
NEB = E // EB
NEBH = EH // EB


@functools.lru_cache(maxsize=None)
def _sc_mesh():
    return plsc.VectorSubcoreMesh(core_axis_name="c", subcore_axis_name="s",
                                  num_cores=NC, num_subcores=NS)


K = 5
NG = NCHG // K
KS = 5
NGS = NCH // KS


@functools.lru_cache(maxsize=None)
def _build_gather_sc():
    half_out = jax.ShapeDtypeStruct((EH, FC), jnp.float32)

    @functools.partial(
        pl.kernel,
        out_type=[half_out, half_out, half_out, half_out],
        mesh=_sc_mesh(),
        scratch_types=[
            [pltpu.VMEM((NCHG, CHG), jnp.int32)] * 4,
            [pltpu.VMEM((CHG, FC), jnp.float32)] * K,
            [pltpu.VMEM((CHG, FC), jnp.float32)] * K,
            [pltpu.SemaphoreType.DMA] * K,
            [pltpu.SemaphoreType.DMA] * K,
            pltpu.SemaphoreType.DMA,
        ],
    )
    def gather_sc(nf_hbm, dst_a, src_a, dst_b, src_b,
                  xi_a, xj_a, xi_b, xj_b,
                  idxs, bufs_i, bufs_j, sems_g, sems_w, sem0):
        wid = lax.axis_index("s") * NC + lax.axis_index("c")
        base = wid * EPWH
        stage = []
        for src_ref, idx_ref in zip((dst_a, src_a, dst_b, src_b), idxs):
            stage.append(pltpu.async_copy(src_ref.at[wid], idx_ref, sem0))
        for cp in stage:
            cp.wait()

        def run_half(idx_d, idx_s, xi_hbm, xj_hbm):
            def body(g, carry):
                t0 = g * K
                fired = []
                for b in range(K):
                    t = t0 + b
                    off = base + t * CHG

                    @pl.when(g > 0)
                    def _(b=b, off=off):
                        pltpu.make_async_copy(
                            bufs_i[b], xi_hbm.at[pl.ds(off, CHG)],
                            sems_w[b]).wait()
                        pltpu.make_async_copy(
                            bufs_j[b], xj_hbm.at[pl.ds(off, CHG)],
                            sems_w[b]).wait()

                    ci = pltpu.async_copy(nf_hbm.at[idx_d.at[t]], bufs_i[b],
                                          sems_g[b])
                    cj = pltpu.async_copy(nf_hbm.at[idx_s.at[t]], bufs_j[b],
                                          sems_g[b])
                    fired.append((ci, cj))
                for b in range(K):
                    t = t0 + b
                    off = base + t * CHG
                    ci, cj = fired[b]
                    ci.wait()
                    cj.wait()
                    pltpu.async_copy(bufs_i[b], xi_hbm.at[pl.ds(off, CHG)],
                                     sems_w[b])
                    pltpu.async_copy(bufs_j[b], xj_hbm.at[pl.ds(off, CHG)],
                                     sems_w[b])
                return carry

            lax.fori_loop(0, NG, body, 0)
            for b in range(K):
                off = base + ((NG - 1) * K + b) * CHG
                pltpu.make_async_copy(
                    bufs_i[b], xi_hbm.at[pl.ds(off, CHG)], sems_w[b]).wait()
                pltpu.make_async_copy(
                    bufs_j[b], xj_hbm.at[pl.ds(off, CHG)], sems_w[b]).wait()

        run_half(idxs[0], idxs[1], xi_a, xj_a)
        run_half(idxs[2], idxs[3], xi_b, xj_b)

    return gather_sc


@functools.lru_cache(maxsize=None)
def _build_gather_half_sc():
    half_out = jax.ShapeDtypeStruct((EH, FC), jnp.float32)

    @functools.partial(
        pl.kernel,
        out_type=[half_out, half_out],
        mesh=_sc_mesh(),
        scratch_types=[
            [pltpu.VMEM((NCHG, CHG), jnp.int32)] * 2,
            [pltpu.VMEM((CHG, FC), jnp.float32)] * K,
            [pltpu.VMEM((CHG, FC), jnp.float32)] * K,
            [pltpu.SemaphoreType.DMA] * K,
            [pltpu.SemaphoreType.DMA] * K,
            pltpu.SemaphoreType.DMA,
        ],
    )
    def gather_half(nf_hbm, dst3, src3, xi_hbm, xj_hbm,
                    idxs, bufs_i, bufs_j, sems_g, sems_w, sem0):
        wid = lax.axis_index("s") * NC + lax.axis_index("c")
        base = wid * EPWH
        cp = pltpu.async_copy(dst3.at[wid], idxs[0], sem0)
        cp2 = pltpu.async_copy(src3.at[wid], idxs[1], sem0)
        cp.wait()
        cp2.wait()

        def body(g, carry):
            t0 = g * K
            fired = []
            for b in range(K):
                t = t0 + b
                off = base + t * CHG

                @pl.when(g > 0)
                def _(b=b, off=off):
                    pltpu.make_async_copy(
                        bufs_i[b], xi_hbm.at[pl.ds(off, CHG)],
                        sems_w[b]).wait()
                    pltpu.make_async_copy(
                        bufs_j[b], xj_hbm.at[pl.ds(off, CHG)],
                        sems_w[b]).wait()

                ci = pltpu.async_copy(nf_hbm.at[idxs[0].at[t]], bufs_i[b],
                                      sems_g[b])
                cj = pltpu.async_copy(nf_hbm.at[idxs[1].at[t]], bufs_j[b],
                                      sems_g[b])
                fired.append((ci, cj))
            for b in range(K):
                t = t0 + b
                off = base + t * CHG
                ci, cj = fired[b]
                ci.wait()
                cj.wait()
                pltpu.async_copy(bufs_i[b], xi_hbm.at[pl.ds(off, CHG)],
                                 sems_w[b])
                pltpu.async_copy(bufs_j[b], xj_hbm.at[pl.ds(off, CHG)],
                                 sems_w[b])
            return carry

        lax.fori_loop(0, NG, body, 0)
        for b in range(K):
            off = base + ((NG - 1) * K + b) * CHG
            pltpu.make_async_copy(
                bufs_i[b], xi_hbm.at[pl.ds(off, CHG)], sems_w[b]).wait()
            pltpu.make_async_copy(
                bufs_j[b], xj_hbm.at[pl.ds(off, CHG)], sems_w[b]).wait()

    return gather_half


def _gather_sc(nf, dst_a, src_a, dst_b, src_b):
    k = _build_gather_half_sc()
    xia, xja = k(nf, dst_a, src_a)
    xib, xjb = k(nf, dst_b, src_b)
    return xia, xja, xib, xjb


@functools.lru_cache(maxsize=None)
def _build_scatter_sc():
    @functools.partial(
        pl.kernel,
        out_type=jax.ShapeDtypeStruct((NC, N, FC), jnp.float32),
        mesh=_sc_mesh(),
        scratch_types=[
            [pltpu.VMEM((KS, CH), jnp.int32)] * 2,
            [pltpu.VMEM((CH, FC), jnp.float32)] * KS,
            [pltpu.SemaphoreType.DMA] * KS,
            [pltpu.SemaphoreType.DMA] * KS,
            [pltpu.SemaphoreType.DMA] * 2,
            pltpu.VMEM_SHARED((N, FC), jnp.float32),
        ],
    )
    def scatter_sc(m_hbm, dst4_hbm, zeros_hbm, out_hbm, idx_bufs, bufs,
                   sems_l, sems_a, sems_i, acc_sh):
        cid = lax.axis_index("c")
        sid = lax.axis_index("s")
        wid = sid * NC + cid
        base = wid * EPW

        row0 = sid * ROWS_PER_TILE
        pltpu.sync_copy(zeros_hbm.at[pl.ds(row0, ROWS_PER_TILE)],
                        acc_sh.at[pl.ds(row0, ROWS_PER_TILE)])

        @pl.when(sid == 0)
        def _():
            tail0 = ROWS_PER_TILE * NS
            pltpu.sync_copy(zeros_hbm.at[pl.ds(tail0, ROWS_TAIL)],
                            acc_sh.at[pl.ds(tail0, ROWS_TAIL)])

        pltpu.async_copy(dst4_hbm.at[wid, 0], idx_bufs[0], sems_i[0]).wait()
        plsc.subcore_barrier()

        def group(g, sel, nsel):
            for b in range(KS):
                @pl.when(g > 0)
                def _(b=b):
                    pltpu.make_async_copy(
                        bufs[b], acc_sh.at[idx_bufs[sel].at[b]],
                        sems_a[b]).wait()

            @pl.when(g + 1 < NGS)
            def _():
                pltpu.async_copy(dst4_hbm.at[wid, g + 1], idx_bufs[nsel],
                                 sems_i[nsel])

            t0 = g * KS
            fired = []
            for b in range(KS):
                off = base + (t0 + b) * CH
                fired.append(pltpu.async_copy(
                    m_hbm.at[pl.ds(off, CH)], bufs[b], sems_l[b]))
            for b in range(KS):
                fired[b].wait()
                pltpu.async_copy(bufs[b], acc_sh.at[idx_bufs[sel].at[b]],
                                 sems_a[b], add=True)

            @pl.when(g + 1 < NGS)
            def _():
                pltpu.make_async_copy(dst4_hbm.at[wid, g + 1],
                                      idx_bufs[nsel], sems_i[nsel]).wait()

        def body(gg, carry):
            group(2 * gg, 0, 1)
            group(2 * gg + 1, 1, 0)
            return carry

        lax.fori_loop(0, NGS // 2, body, 0)
        for b in range(KS):
            pltpu.make_async_copy(
                bufs[b], acc_sh.at[idx_bufs[1].at[b]], sems_a[b]).wait()

        plsc.subcore_barrier()
        pltpu.sync_copy(acc_sh.at[pl.ds(row0, ROWS_PER_TILE)],
                        out_hbm.at[cid, pl.ds(row0, ROWS_PER_TILE)])

        @pl.when(sid == 0)
        def _():
            tail0 = ROWS_PER_TILE * NS
            pltpu.sync_copy(acc_sh.at[pl.ds(tail0, ROWS_TAIL)],
                            out_hbm.at[cid, pl.ds(tail0, ROWS_TAIL)])

    return scatter_sc


def _scatter_sc(m, dst4, zeros):
    return _build_scatter_sc()(m, dst4, zeros)


def _dot(a, b):
    return jnp.dot(a, b, preferred_element_type=jnp.float32)


def _prep_body(x_ref, w_ref, b_ref, nf_ref):
    nf_ref[...] = _dot(x_ref[...], w_ref[...]) + b_ref[...]


def _prep_tc(x, w_atom, b_atom):
    return pl.pallas_call(
        _prep_body,
        out_shape=jax.ShapeDtypeStruct((N, FC), jnp.float32),
    )(x, w_atom, b_atom)


def _edge_h(ea, xi, xj, wei, bei, weu, beu, wemb, bemb):
    centers = (lax.broadcasted_iota(jnp.int32, (1, BINS), 1)
               .astype(jnp.float32) * (8.0 / (BINS - 1)))
    gamma = (BINS - 1) / 8.0
    r = jnp.exp(-gamma * (ea[:, None] - centers) ** 2)
    a1, a2, a3 = wei[0:FC], wei[FC:2 * FC], wei[2 * FC:3 * FC]
    c1, c2, c3 = weu[0:FC], weu[FC:2 * FC], weu[2 * FC:3 * FC]
    weg = _dot(wemb, a3)
    weu_g = _dot(wemb, c3)
    cg = bei + _dot(bemb, a3)
    cu = beu + _dot(bemb, c3)
    hg = _dot(xi, a1) + _dot(xj, a2) + _dot(r, weg) + cg
    hu = _dot(xi, c1) + _dot(xj, c2) + _dot(r, weu_g) + cu
    return hg, hu


def _stats_body(ea_ref, xi_ref, xj_ref, wei_ref, bei_ref, weu_ref, beu_ref,
                wemb_ref, bemb_ref, stats_ref):
    hg, hu = _edge_h(ea_ref[0, 0, :], xi_ref[...], xj_ref[...],
                     wei_ref[...], bei_ref[...], weu_ref[...], beu_ref[...],
                     wemb_ref[...], bemb_ref[...])
    blk = jnp.stack([
        jnp.sum(hg, axis=0), jnp.sum(hg * hg, axis=0),
        jnp.sum(hu, axis=0), jnp.sum(hu * hu, axis=0),
    ], axis=0)

    @pl.when(pl.program_id(0) == 0)
    def _():
        stats_ref[...] = jnp.zeros_like(stats_ref)

    stats_ref[...] += blk


def _edge_stats_tc(ea3, xi, xj, wei, bei, weu, beu, wemb, bemb):
    wspec = pl.BlockSpec
    return pl.pallas_call(
        _stats_body,
        grid=(NEBH,),
        in_specs=[
            wspec((1, 1, EB), lambda i: (i, 0, 0)),
            wspec((EB, FC), lambda i: (i, 0)),
            wspec((EB, FC), lambda i: (i, 0)),
            wspec((3 * FC, FC), lambda i: (0, 0)),
            wspec((1, FC), lambda i: (0, 0)),
            wspec((3 * FC, FC), lambda i: (0, 0)),
            wspec((1, FC), lambda i: (0, 0)),
            wspec((BINS, FC), lambda i: (0, 0)),
            wspec((1, FC), lambda i: (0, 0)),
        ],
        out_specs=wspec((4, FC), lambda i: (0, 0)),
        out_shape=jax.ShapeDtypeStruct((4, FC), jnp.float32),
    )(ea3, xi, xj, wei, bei, weu, beu, wemb, bemb)


def _sigmoid(v):
    return 1.0 / (1.0 + jnp.exp(-v))


def _softplus(v):
    return jnp.maximum(v, 0.0) + jnp.log1p(jnp.exp(-jnp.abs(v)))


def _m_body(m_in_ref, ea_ref, xi_ref, xj_ref, sa_ref, sb_ref, wei_ref,
            bei_ref, weu_ref, beu_ref, wemb_ref, bemb_ref, gg_ref, gb_ref,
            ug_ref, ub_ref, m_ref):
    del m_in_ref
    hg, hu = _edge_h(ea_ref[0, 0, :], xi_ref[...], xj_ref[...],
                     wei_ref[...], bei_ref[...], weu_ref[...], beu_ref[...],
                     wemb_ref[...], bemb_ref[...])
    s = sa_ref[...] + sb_ref[...]
    inv_e = 1.0 / E
    mu_g = s[0:1, :] * inv_e
    var_g = s[1:2, :] * inv_e - mu_g * mu_g
    mu_u = s[2:3, :] * inv_e
    var_u = s[3:4, :] * inv_e - mu_u * mu_u
    sc_g = gg_ref[...] * lax.rsqrt(var_g + 1e-5)
    sc_u = ug_ref[...] * lax.rsqrt(var_u + 1e-5)
    gate = _sigmoid((hg - mu_g) * sc_g + gb_ref[...])
    upd = _softplus((hu - mu_u) * sc_u + ub_ref[...])
    m_ref[...] = gate * upd


def _edge_m_tc(half, m_in, ea3, xi, xj, sa, sb, wei, bei, weu, beu,
               wemb, bemb, gg, gb, ug, ub):
    wspec = pl.BlockSpec
    base = half * NEBH
    if m_in is None:
        body = functools.partial(_m_body, None)
        first_specs, first_args, aliases = [], (), {}
    else:
        body = _m_body
        first_specs = [wspec(memory_space=pl.ANY)]
        first_args = (m_in,)
        aliases = {0: 0}
    return pl.pallas_call(
        body,
        grid=(NEBH,),
        in_specs=first_specs + [
            wspec((1, 1, EB), lambda i: (i, 0, 0)),
            wspec((EB, FC), lambda i: (i, 0)),
            wspec((EB, FC), lambda i: (i, 0)),
            wspec((4, FC), lambda i: (0, 0)),
            wspec((4, FC), lambda i: (0, 0)),
            wspec((3 * FC, FC), lambda i: (0, 0)),
            wspec((1, FC), lambda i: (0, 0)),
            wspec((3 * FC, FC), lambda i: (0, 0)),
            wspec((1, FC), lambda i: (0, 0)),
            wspec((BINS, FC), lambda i: (0, 0)),
            wspec((1, FC), lambda i: (0, 0)),
            wspec((1, FC), lambda i: (0, 0)),
            wspec((1, FC), lambda i: (0, 0)),
            wspec((1, FC), lambda i: (0, 0)),
            wspec((1, FC), lambda i: (0, 0)),
        ],
        out_specs=wspec((EB, FC), lambda i: (i + base, 0)),
        out_shape=jax.ShapeDtypeStruct((E, FC), jnp.float32),
        input_output_aliases=aliases,
    )(*first_args, ea3, xi, xj, sa, sb, wei, bei, weu, beu, wemb, bemb,
      gg, gb, ug, ub)


def _final_body(nf_ref, aggp_ref, batch_ref, wat_ref, bat_ref, bng_ref,
                bnb_ref, wfc_ref, bfc_ref, wout_ref, bout_ref, out_ref):
    agg = aggp_ref[0] + aggp_ref[1]
    mu = jnp.mean(agg, axis=0, keepdims=True)
    var = jnp.mean(agg * agg, axis=0, keepdims=True) - mu * mu
    bn = (agg - mu) * lax.rsqrt(var + 1e-5) * bng_ref[...] + bnb_ref[...]
    nf2 = _dot(nf_ref[...] + bn, wat_ref[...]) + bat_ref[...]

    gids = lax.broadcasted_iota(jnp.int32, (1, G), 1)
    onehot = (batch_ref[...] == gids).astype(jnp.float32)
    counts = jnp.sum(onehot, axis=0)
    pooled_sum = lax.dot_general(onehot, nf2, (((0,), (0,)), ((), ())),
                                 preferred_element_type=jnp.float32)
    pooled = pooled_sum / jnp.maximum(counts, 1.0)[:, None]
    feats = _dot(pooled, wfc_ref[...]) + bfc_ref[...]
    feats = feats * _sigmoid(feats)
    out_ref[...] = _dot(feats, wout_ref[...]) + bout_ref[...]


def _final_tc(nf, aggp, batch2, wat, bat, bng, bnb, wfc, bfc, wout, bout):
    return pl.pallas_call(
        _final_body,
        out_shape=jax.ShapeDtypeStruct((G, 1), jnp.float32),
    )(nf, aggp, batch2, wat, bat, bng, bnb, wfc, bfc, wout, bout)


def kernel(x, edge_attr, edge_index, batch,
           W_atom_emb, b_atom_emb, W_edge_emb, b_edge_emb,
           W_atoms, b_atoms, bn_g, bn_b,
           W_ei, b_ei, bn_ei_g, bn_ei_b,
           W_eu, b_eu, bn_eu_g, bn_eu_b,
           W_fc, b_fc, W_out, b_out):
    src = edge_index[0]
    dst = edge_index[1]
    dst4 = dst.reshape(NW, NGS, KS, CH)
    halves = []
    for h in range(2):
        d3 = lax.slice_in_dim(dst, h * EH, (h + 1) * EH).reshape(
            NW, NCHG, CHG)
        s3 = lax.slice_in_dim(src, h * EH, (h + 1) * EH).reshape(
            NW, NCHG, CHG)
        ea3 = lax.slice_in_dim(edge_attr, h * EH, (h + 1) * EH).reshape(
            NEBH, 1, EB)
        halves.append((d3, s3, ea3))
    r1 = lambda v: v.reshape(1, FC)

    nf = _prep_tc(x, W_atom_emb, b_atom_emb.reshape(1, FC))
    xia, xja, xib, xjb = _gather_sc(nf, halves[0][0], halves[0][1],
                                    halves[1][0], halves[1][1])
    gathered = [(xia, xja), (xib, xjb)]

    wargs = (W_ei, r1(b_ei), W_eu, r1(b_eu), W_edge_emb, r1(b_edge_emb))
    stats = [_edge_stats_tc(halves[h][2], *gathered[h], *wargs)
             for h in range(2)]

    bnargs = (r1(bn_ei_g), r1(bn_ei_b), r1(bn_eu_g), r1(bn_eu_b))
    m = None
    for h in range(2):
        m = _edge_m_tc(h, m, halves[h][2], *gathered[h], stats[0], stats[1],
                       *wargs, *bnargs)

    zeros = jnp.zeros((N, FC), jnp.float32)
    aggp = _scatter_sc(m, dst4, zeros)

    out = _final_tc(nf, aggp, batch.reshape(N, 1), W_atoms,
                    r1(b_atoms), r1(bn_g), r1(bn_b), W_fc, r1(b_fc),
                    W_out, b_out.reshape(1, 1))
    return jnp.squeeze(out)
